# Initial kernel scaffold; baseline (speedup 1.0000x reference)
import jax
import jax.numpy as jnp
from jax.experimental import pallas as pl


def kernel(x, edge_index, edge_features, W1, b1, W2, b2):
    raise NotImplementedError("write your pallas kernel here")


# speedup vs baseline: 4.0418x; 4.0418x over previous
import functools

import jax
import jax.numpy as jnp
from jax import lax
from jax.experimental import pallas as pl
from jax.experimental.pallas import tpu as pltpu
from jax.experimental.pallas import tpu_sc as plsc

_N, _E, _D, _DE, _H = 10000, 320000, 128, 16, 16

_NC, _NS = 2, 16
_NW = _NC * _NS
_CH = 1280
_NCHUNK = _E // _CH
_TMAX = -(-_NCHUNK // _NW)
_SUB = 128
_NSUB = _CH // _SUB
_CROWS = _CH // 8


def _ab_body(x_ref, wa_ref, wb_ref, a_ref, b_ref):
    x = x_ref[...]
    a_ref[...] = jnp.dot(x, wa_ref[...], preferred_element_type=jnp.float32)
    b_ref[...] = jnp.dot(x, wb_ref[...], preferred_element_type=jnp.float32)


_ab_call = pl.pallas_call(
    _ab_body,
    out_shape=[
        jax.ShapeDtypeStruct((_N, _H), jnp.float32),
        jax.ShapeDtypeStruct((_N, _H), jnp.float32),
    ],
)

_BC = 4000


def _c_body(ef_ref, w_ref, b_ref, c_ref):
    c_ref[...] = (
        jnp.dot(ef_ref[...], w_ref[...], preferred_element_type=jnp.float32)
        + b_ref[...]
    )


_c_call = pl.pallas_call(
    _c_body,
    grid=(_E // 8 // _BC,),
    in_specs=[
        pl.BlockSpec((_BC, 128), lambda i: (i, 0)),
        pl.BlockSpec((128, 128), lambda i: (0, 0)),
        pl.BlockSpec((1, 128), lambda i: (0, 0)),
    ],
    out_specs=pl.BlockSpec((_BC, 128), lambda i: (i, 0)),
    out_shape=jax.ShapeDtypeStruct((_E // 8, 128), jnp.float32),
)

_mesh = plsc.VectorSubcoreMesh(
    core_axis_name="core", subcore_axis_name="subcore",
    num_cores=_NC, num_subcores=_NS,
)


@functools.partial(
    pl.kernel,
    out_type=jax.ShapeDtypeStruct((_E,), jnp.float32),
    mesh=_mesh,
    compiler_params=pltpu.CompilerParams(
        needs_layout_passes=False, use_tc_tiling_on_sc=False),
    scratch_types=[
        pltpu.VMEM((_CH,), jnp.int32),
        pltpu.VMEM((_CH,), jnp.int32),
        pltpu.VMEM((_SUB, _H), jnp.float32),
        pltpu.VMEM((_SUB, _H), jnp.float32),
        pltpu.VMEM((_CROWS, 128), jnp.float32),
        pltpu.VMEM((_CH,), jnp.float32),
        pltpu.VMEM((_H,), jnp.float32),
        pltpu.VMEM((_H,), jnp.float32),
        pltpu.SemaphoreType.DMA,
        pltpu.SemaphoreType.DMA,
    ],
)
def _sc_edge(a_hbm, b_hbm, c_hbm, og_hbm, dg_hbm, w2_hbm, b2_hbm, out_hbm,
             io_v, id_v, ra_v, rb_v, rc_v, o_v, w2_v, b2_v, sem_a, sem_b):
    wid = lax.axis_index("subcore") * _NC + lax.axis_index("core")
    pltpu.sync_copy(w2_hbm, w2_v)
    pltpu.sync_copy(b2_hbm, b2_v)
    w2vec = w2_v[:]
    b2vec = b2_v[:]
    lane = lax.iota(jnp.int32, 16)
    last = lane == 15

    def chunk_body(t, carry):
        ci = wid + t * _NW

        @pl.when(ci < _NCHUNK)
        def _():
            cbase = pl.multiple_of(ci * _CH, _CH)
            crow = pl.multiple_of(ci * _CROWS, _CROWS)
            pltpu.sync_copy(og_hbm.at[pl.ds(cbase, _CH)], io_v)
            pltpu.sync_copy(dg_hbm.at[pl.ds(cbase, _CH)], id_v)
            pltpu.sync_copy(c_hbm.at[pl.ds(crow, _CROWS)], rc_v)

            def sub_body(s, scarry):
                sl = pl.ds(s * _SUB, _SUB)
                ca = pltpu.async_copy(a_hbm.at[io_v.at[sl]], ra_v, sem_a)
                cb = pltpu.async_copy(b_hbm.at[id_v.at[sl]], rb_v, sem_b)
                ca.wait()
                cb.wait()

                def edge_body(l8, ecarry):
                    for dl in range(8):
                        l = l8 * 8 + dl
                        e = s * _SUB + l
                        va = ra_v[l, :]
                        vb = rb_v[l, :]
                        vc = rc_v[e // 8, pl.ds((e % 8) * 16, 16)]
                        h = va + vb + vc
                        h = jnp.maximum(h, h * 0.01)
                        csum = plsc.cumsum(h * w2vec)
                        val = csum + b2vec
                        val = jnp.maximum(val, val * 0.01)
                        plsc.store_scatter(
                            o_v, [jnp.full((16,), e, jnp.int32)], val,
                            mask=last)
                    return ecarry

                lax.fori_loop(0, _SUB // 8, edge_body, 0)
                return scarry

            lax.fori_loop(0, _NSUB, sub_body, 0)
            pltpu.sync_copy(o_v, out_hbm.at[pl.ds(cbase, _CH)])

        return carry

    lax.fori_loop(0, _TMAX, chunk_body, 0)


def kernel(x, edge_index, edge_features, W1, b1, W2, b2):
    wa = W1[:_D]
    wb = W1[_D:2 * _D]
    we = W1[2 * _D:]
    a, b = _ab_call(x, wa, wb)
    ef_r = edge_features.reshape(_E // 8, 8 * _DE)
    we_blk = jnp.kron(jnp.eye(8, dtype=jnp.float32), we)
    c = _c_call(ef_r, we_blk, jnp.tile(b1, 8).reshape(1, 128))
    return _sc_edge(a, b, c, edge_index[0], edge_index[1],
                    W2.reshape(_H), jnp.broadcast_to(b2, (_H,)))


# speedup vs baseline: 6.3493x; 1.5709x over previous
import functools

import jax
import jax.numpy as jnp
from jax import lax
from jax.experimental import pallas as pl
from jax.experimental.pallas import tpu as pltpu
from jax.experimental.pallas import tpu_sc as plsc

_N, _E, _D, _DE, _H = 10000, 320000, 128, 16, 16

_NC, _NS = 2, 16
_NW = _NC * _NS
_CH = 1280
_NCHUNK = _E // _CH
_TMAX = -(-_NCHUNK // _NW)
_SUB = 128
_NSUB = _CH // _SUB
_CROWS = _CH // 8


def _ab_body(x_ref, wa_ref, wb_ref, a_ref, b_ref):
    x = x_ref[...]
    a_ref[...] = jnp.dot(x, wa_ref[...], preferred_element_type=jnp.float32)
    b_ref[...] = jnp.dot(x, wb_ref[...], preferred_element_type=jnp.float32)


_ab_call = pl.pallas_call(
    _ab_body,
    out_shape=[
        jax.ShapeDtypeStruct((_N, _H), jnp.float32),
        jax.ShapeDtypeStruct((_N, _H), jnp.float32),
    ],
)

_BC = 4000


def _c_body(ef_ref, w_ref, b_ref, c_ref):
    c_ref[...] = (
        jnp.dot(ef_ref[...], w_ref[...], preferred_element_type=jnp.float32)
        + b_ref[...]
    )


_c_call = pl.pallas_call(
    _c_body,
    grid=(_E // 8 // _BC,),
    in_specs=[
        pl.BlockSpec((_BC, 128), lambda i: (i, 0)),
        pl.BlockSpec((128, 128), lambda i: (0, 0)),
        pl.BlockSpec((1, 128), lambda i: (0, 0)),
    ],
    out_specs=pl.BlockSpec((_BC, 128), lambda i: (i, 0)),
    out_shape=jax.ShapeDtypeStruct((_E // 8, 128), jnp.float32),
)

_mesh = plsc.VectorSubcoreMesh(
    core_axis_name="core", subcore_axis_name="subcore",
    num_cores=_NC, num_subcores=_NS,
)


@functools.partial(
    pl.kernel,
    out_type=jax.ShapeDtypeStruct((_E,), jnp.float32),
    mesh=_mesh,
    compiler_params=pltpu.CompilerParams(
        needs_layout_passes=False, use_tc_tiling_on_sc=False),
    scratch_types=[
        pltpu.VMEM((_CH,), jnp.int32),
        pltpu.VMEM((_CH,), jnp.int32),
        pltpu.VMEM((_SUB, _H), jnp.float32),
        pltpu.VMEM((_SUB, _H), jnp.float32),
        pltpu.VMEM((_SUB, _H), jnp.float32),
        pltpu.VMEM((_SUB, _H), jnp.float32),
        pltpu.VMEM((_CROWS, 128), jnp.float32),
        pltpu.VMEM((_CH,), jnp.float32),
        pltpu.VMEM((_H, _H), jnp.float32),
        pltpu.VMEM((_H,), jnp.float32),
        pltpu.SemaphoreType.DMA,
        pltpu.SemaphoreType.DMA,
        pltpu.SemaphoreType.DMA,
    ],
)
def _sc_edge(a_hbm, b_hbm, c_hbm, og_hbm, dg_hbm, w2_hbm, b2_hbm, out_hbm,
             io_v, id_v, ra0_v, ra1_v, rb0_v, rb1_v, rc_v, o_v, w2_v, b2_v,
             sem_a, sem_b, sem_c):
    wid = lax.axis_index("subcore") * _NC + lax.axis_index("core")
    pltpu.sync_copy(w2_hbm, w2_v)
    pltpu.sync_copy(b2_hbm, b2_v)
    w2rows = [w2_v[j, :] for j in range(_H)]
    b2vec = b2_v[:]
    lane = lax.iota(jnp.int32, 16)
    rowoff = lane // 8
    ccols = [(lane % 8) * 16 + j for j in range(_H)]
    rabufs = [ra0_v, ra1_v]
    rbbufs = [rb0_v, rb1_v]

    def chunk_body(t, carry):
        ci = wid + t * _NW

        @pl.when(ci < _NCHUNK)
        def _():
            cbase = pl.multiple_of(ci * _CH, _CH)
            crow = pl.multiple_of(ci * _CROWS, _CROWS)
            pltpu.sync_copy(og_hbm.at[pl.ds(cbase, _CH)], io_v)
            pltpu.sync_copy(dg_hbm.at[pl.ds(cbase, _CH)], id_v)
            cc = pltpu.async_copy(c_hbm.at[pl.ds(crow, _CROWS)], rc_v, sem_c)

            def fire(s):
                sl = pl.ds(s * _SUB, _SUB)
                ca = pltpu.async_copy(
                    a_hbm.at[io_v.at[sl]], rabufs[s % 2], sem_a)
                cb = pltpu.async_copy(
                    b_hbm.at[id_v.at[sl]], rbbufs[s % 2], sem_b)
                return ca, cb

            desc = {0: fire(0)}
            cc.wait()
            for s in range(_NSUB):
                if s + 1 < _NSUB:
                    desc[s + 1] = fire(s + 1)
                ca, cb = desc.pop(s)
                ca.wait()
                cb.wait()
                ra_v = rabufs[s % 2]
                rb_v = rbbufs[s % 2]

                def group_body(gg, gcarry, s=s, ra_v=ra_v, rb_v=rb_v):
                    ridx = gg * 16 + lane
                    crow16 = s * 16 + gg * 2 + rowoff
                    acc = b2vec
                    for j in range(_H):
                        cidx = jnp.full((16,), j, jnp.int32)
                        va = plsc.load_gather(ra_v, [ridx, cidx])
                        vb = plsc.load_gather(rb_v, [ridx, cidx])
                        vc = plsc.load_gather(rc_v, [crow16, ccols[j]])
                        h = va + vb + vc
                        acc = acc + jnp.maximum(h, h * 0.01) * w2rows[j]
                    o_v[pl.ds(s * _SUB + gg * 16, 16)] = (
                        jnp.maximum(acc, acc * 0.01))
                    return gcarry

                lax.fori_loop(0, _SUB // 16, group_body, 0)

            pltpu.sync_copy(o_v, out_hbm.at[pl.ds(cbase, _CH)])

        return carry

    lax.fori_loop(0, _TMAX, chunk_body, 0)


def kernel(x, edge_index, edge_features, W1, b1, W2, b2):
    wa = W1[:_D]
    wb = W1[_D:2 * _D]
    we = W1[2 * _D:]
    a, b = _ab_call(x, wa, wb)
    ef_r = edge_features.reshape(_E // 8, 8 * _DE)
    we_blk = jnp.kron(jnp.eye(8, dtype=jnp.float32), we)
    c = _c_call(ef_r, we_blk, jnp.tile(b1, 8).reshape(1, 128))
    return _sc_edge(a, b, c, edge_index[0], edge_index[1],
                    jnp.broadcast_to(W2, (_H, _H)), jnp.broadcast_to(b2, (_H,)))


# speedup vs baseline: 6.3825x; 1.0052x over previous
import functools

import jax
import jax.numpy as jnp
from jax import lax
from jax.experimental import pallas as pl
from jax.experimental.pallas import tpu as pltpu
from jax.experimental.pallas import tpu_sc as plsc

_N, _E, _D, _DE, _H = 10000, 320000, 128, 16, 16

_NC, _NS = 2, 16
_NW = _NC * _NS
_CH = 1280
_NCHUNK = _E // _CH
_TMAX = -(-_NCHUNK // _NW)
_SUB = 320
_NSUB = _CH // _SUB
_CROWS = _CH // 8


def _ab_body(x_ref, wa_ref, wb_ref, a_ref, b_ref):
    x = x_ref[...]
    a_ref[...] = jnp.dot(x, wa_ref[...], preferred_element_type=jnp.float32)
    b_ref[...] = jnp.dot(x, wb_ref[...], preferred_element_type=jnp.float32)


_ab_call = pl.pallas_call(
    _ab_body,
    out_shape=[
        jax.ShapeDtypeStruct((_N, _H), jnp.float32),
        jax.ShapeDtypeStruct((_N, _H), jnp.float32),
    ],
)

_BC = 4000


def _c_body(ef_ref, w_ref, b_ref, c_ref):
    c_ref[...] = (
        jnp.dot(ef_ref[...], w_ref[...], preferred_element_type=jnp.float32)
        + b_ref[...]
    )


_c_call = pl.pallas_call(
    _c_body,
    grid=(_E // 8 // _BC,),
    in_specs=[
        pl.BlockSpec((_BC, 128), lambda i: (i, 0)),
        pl.BlockSpec((128, 128), lambda i: (0, 0)),
        pl.BlockSpec((1, 128), lambda i: (0, 0)),
    ],
    out_specs=pl.BlockSpec((_BC, 128), lambda i: (i, 0)),
    out_shape=jax.ShapeDtypeStruct((_E // 8, 128), jnp.float32),
)

_mesh = plsc.VectorSubcoreMesh(
    core_axis_name="core", subcore_axis_name="subcore",
    num_cores=_NC, num_subcores=_NS,
)


@functools.partial(
    pl.kernel,
    out_type=jax.ShapeDtypeStruct((_E,), jnp.float32),
    mesh=_mesh,
    compiler_params=pltpu.CompilerParams(
        needs_layout_passes=False, use_tc_tiling_on_sc=False),
    scratch_types=[
        pltpu.VMEM((_CH,), jnp.int32),
        pltpu.VMEM((_CH,), jnp.int32),
        pltpu.VMEM((_SUB, _H), jnp.float32),
        pltpu.VMEM((_SUB, _H), jnp.float32),
        pltpu.VMEM((_SUB, _H), jnp.float32),
        pltpu.VMEM((_SUB, _H), jnp.float32),
        pltpu.VMEM((_CROWS, 128), jnp.float32),
        pltpu.VMEM((_CH,), jnp.float32),
        pltpu.VMEM((_H, _H), jnp.float32),
        pltpu.VMEM((_H,), jnp.float32),
        pltpu.SemaphoreType.DMA,
        pltpu.SemaphoreType.DMA,
        pltpu.SemaphoreType.DMA,
    ],
)
def _sc_edge(a_hbm, b_hbm, c_hbm, og_hbm, dg_hbm, w2_hbm, b2_hbm, out_hbm,
             io_v, id_v, ra0_v, ra1_v, rb0_v, rb1_v, rc_v, o_v, w2_v, b2_v,
             sem_a, sem_b, sem_c):
    wid = lax.axis_index("subcore") * _NC + lax.axis_index("core")
    pltpu.sync_copy(w2_hbm, w2_v)
    pltpu.sync_copy(b2_hbm, b2_v)
    w2rows = [w2_v[j, :] for j in range(_H)]
    b2vec = b2_v[:]
    lane = lax.iota(jnp.int32, 16)
    rowoff = lane // 8
    ccols = [(lane % 8) * 16 + j for j in range(_H)]
    rabufs = [ra0_v, ra1_v]
    rbbufs = [rb0_v, rb1_v]

    def chunk_body(t, carry):
        ci = wid + t * _NW

        @pl.when(ci < _NCHUNK)
        def _():
            cbase = pl.multiple_of(ci * _CH, _CH)
            crow = pl.multiple_of(ci * _CROWS, _CROWS)
            pltpu.sync_copy(og_hbm.at[pl.ds(cbase, _CH)], io_v)
            pltpu.sync_copy(dg_hbm.at[pl.ds(cbase, _CH)], id_v)
            cc = pltpu.async_copy(c_hbm.at[pl.ds(crow, _CROWS)], rc_v, sem_c)

            def fire(s):
                sl = pl.ds(s * _SUB, _SUB)
                ca = pltpu.async_copy(
                    a_hbm.at[io_v.at[sl]], rabufs[s % 2], sem_a)
                cb = pltpu.async_copy(
                    b_hbm.at[id_v.at[sl]], rbbufs[s % 2], sem_b)
                return ca, cb

            desc = {0: fire(0)}
            cc.wait()
            for s in range(_NSUB):
                if s + 1 < _NSUB:
                    desc[s + 1] = fire(s + 1)
                ca, cb = desc.pop(s)
                ca.wait()
                cb.wait()
                ra_v = rabufs[s % 2]
                rb_v = rbbufs[s % 2]

                def group_body(gg, gcarry, s=s, ra_v=ra_v, rb_v=rb_v):
                    ridx = gg * 16 + lane
                    crow16 = s * 16 + gg * 2 + rowoff
                    acc = b2vec
                    for j in range(_H):
                        cidx = jnp.full((16,), j, jnp.int32)
                        va = plsc.load_gather(ra_v, [ridx, cidx])
                        vb = plsc.load_gather(rb_v, [ridx, cidx])
                        vc = plsc.load_gather(rc_v, [crow16, ccols[j]])
                        h = va + vb + vc
                        acc = acc + jnp.maximum(h, h * 0.01) * w2rows[j]
                    o_v[pl.ds(s * _SUB + gg * 16, 16)] = (
                        jnp.maximum(acc, acc * 0.01))
                    return gcarry

                lax.fori_loop(0, _SUB // 16, group_body, 0)

            pltpu.sync_copy(o_v, out_hbm.at[pl.ds(cbase, _CH)])

        return carry

    lax.fori_loop(0, _TMAX, chunk_body, 0)


def kernel(x, edge_index, edge_features, W1, b1, W2, b2):
    wa = W1[:_D]
    wb = W1[_D:2 * _D]
    we = W1[2 * _D:]
    a, b = _ab_call(x, wa, wb)
    ef_r = edge_features.reshape(_E // 8, 8 * _DE)
    we_blk = jnp.kron(jnp.eye(8, dtype=jnp.float32), we)
    c = _c_call(ef_r, we_blk, jnp.tile(b1, 8).reshape(1, 128))
    return _sc_edge(a, b, c, edge_index[0], edge_index[1],
                    jnp.broadcast_to(W2, (_H, _H)), jnp.broadcast_to(b2, (_H,)))


# speedup vs baseline: 6.5854x; 1.0318x over previous
import functools

import jax
import jax.numpy as jnp
from jax import lax
from jax.experimental import pallas as pl
from jax.experimental.pallas import tpu as pltpu
from jax.experimental.pallas import tpu_sc as plsc

_N, _E, _D, _DE, _H = 10000, 320000, 128, 16, 16

_NC, _NS = 2, 16
_NW = _NC * _NS
_CH = 1280
_NCHUNK = _E // _CH
_TMAX = -(-_NCHUNK // _NW)
_SUB = 128
_NSUB = _CH // _SUB
_CROWS = _CH // 8


def _ab_body(x_ref, wa_ref, wb_ref, a_ref, b_ref):
    x = x_ref[...]
    a_ref[...] = jnp.dot(x, wa_ref[...], preferred_element_type=jnp.float32)
    b_ref[...] = jnp.dot(x, wb_ref[...], preferred_element_type=jnp.float32)


_ab_call = pl.pallas_call(
    _ab_body,
    out_shape=[
        jax.ShapeDtypeStruct((_N, _H), jnp.float32),
        jax.ShapeDtypeStruct((_N, _H), jnp.float32),
    ],
)

_BC = 4000


def _c_body(ef_ref, w_ref, b_ref, c_ref):
    c_ref[...] = (
        jnp.dot(ef_ref[...], w_ref[...], preferred_element_type=jnp.float32)
        + b_ref[...]
    )


_c_call = pl.pallas_call(
    _c_body,
    grid=(_E // 8 // _BC,),
    in_specs=[
        pl.BlockSpec((_BC, 128), lambda i: (i, 0)),
        pl.BlockSpec((128, 128), lambda i: (0, 0)),
        pl.BlockSpec((1, 128), lambda i: (0, 0)),
    ],
    out_specs=pl.BlockSpec((_BC, 128), lambda i: (i, 0)),
    out_shape=jax.ShapeDtypeStruct((_E // 8, 128), jnp.float32),
)

_mesh = plsc.VectorSubcoreMesh(
    core_axis_name="core", subcore_axis_name="subcore",
    num_cores=_NC, num_subcores=_NS,
)


@functools.partial(
    pl.kernel,
    out_type=jax.ShapeDtypeStruct((_E,), jnp.float32),
    mesh=_mesh,
    compiler_params=pltpu.CompilerParams(
        needs_layout_passes=False, use_tc_tiling_on_sc=False),
    scratch_types=[
        pltpu.VMEM((_CH,), jnp.int32),
        pltpu.VMEM((_CH,), jnp.int32),
        pltpu.VMEM((_SUB, _H), jnp.float32),
        pltpu.VMEM((_SUB, _H), jnp.float32),
        pltpu.VMEM((_SUB, _H), jnp.float32),
        pltpu.VMEM((_SUB, _H), jnp.float32),
        pltpu.VMEM((_CROWS, 128), jnp.float32),
        pltpu.VMEM((_CH,), jnp.float32),
        pltpu.VMEM((_H, _H), jnp.float32),
        pltpu.VMEM((_H,), jnp.float32),
        pltpu.SemaphoreType.DMA,
        pltpu.SemaphoreType.DMA,
        pltpu.SemaphoreType.DMA,
    ],
)
def _sc_edge(a_hbm, b_hbm, c_hbm, ei_hbm, w2_hbm, b2_hbm, out_hbm,
             io_v, id_v, ra0_v, ra1_v, rb0_v, rb1_v, rc_v, o_v, w2_v, b2_v,
             sem_a, sem_b, sem_c):
    wid = lax.axis_index("subcore") * _NC + lax.axis_index("core")
    pltpu.sync_copy(w2_hbm, w2_v)
    pltpu.sync_copy(b2_hbm, b2_v)
    w2rows = [w2_v[j, :] for j in range(_H)]
    b2vec = b2_v[:]
    lane = lax.iota(jnp.int32, 16)
    rowoff = lane // 8
    ccols = [(lane % 8) * 16 + j for j in range(_H)]
    rabufs = [ra0_v, ra1_v]
    rbbufs = [rb0_v, rb1_v]

    def chunk_body(t, carry):
        ci = wid + t * _NW

        @pl.when(ci < _NCHUNK)
        def _():
            cbase = pl.multiple_of(ci * _CH, _CH)
            crow = pl.multiple_of(ci * _CROWS, _CROWS)
            pltpu.sync_copy(ei_hbm.at[pl.ds(cbase, _CH)], io_v)
            pltpu.sync_copy(ei_hbm.at[pl.ds(_E + cbase, _CH)], id_v)
            cc = pltpu.async_copy(c_hbm.at[pl.ds(crow, _CROWS)], rc_v, sem_c)

            def fire(s):
                sl = pl.ds(s * _SUB, _SUB)
                ca = pltpu.async_copy(
                    a_hbm.at[io_v.at[sl]], rabufs[s % 2], sem_a)
                cb = pltpu.async_copy(
                    b_hbm.at[id_v.at[sl]], rbbufs[s % 2], sem_b)
                return ca, cb

            desc = {0: fire(0)}
            cc.wait()
            for s in range(_NSUB):
                if s + 1 < _NSUB:
                    desc[s + 1] = fire(s + 1)
                ca, cb = desc.pop(s)
                ca.wait()
                cb.wait()
                ra_v = rabufs[s % 2]
                rb_v = rbbufs[s % 2]

                def group_body(gg, gcarry, s=s, ra_v=ra_v, rb_v=rb_v):
                    ridx = gg * 16 + lane
                    crow16 = s * 16 + gg * 2 + rowoff
                    acc = b2vec
                    for j in range(_H):
                        cidx = jnp.full((16,), j, jnp.int32)
                        va = plsc.load_gather(ra_v, [ridx, cidx])
                        vb = plsc.load_gather(rb_v, [ridx, cidx])
                        vc = plsc.load_gather(rc_v, [crow16, ccols[j]])
                        h = va + vb + vc
                        acc = acc + jnp.maximum(h, h * 0.01) * w2rows[j]
                    o_v[pl.ds(s * _SUB + gg * 16, 16)] = (
                        jnp.maximum(acc, acc * 0.01))
                    return gcarry

                lax.fori_loop(0, _SUB // 16, group_body, 0)

            pltpu.sync_copy(o_v, out_hbm.at[pl.ds(cbase, _CH)])

        return carry

    lax.fori_loop(0, _TMAX, chunk_body, 0)


def kernel(x, edge_index, edge_features, W1, b1, W2, b2):
    wa = W1[:_D]
    wb = W1[_D:2 * _D]
    we = W1[2 * _D:]
    a, b = _ab_call(x, wa, wb)
    ef_r = edge_features.reshape(_E // 8, 8 * _DE)
    we_blk = jnp.kron(jnp.eye(8, dtype=jnp.float32), we)
    c = _c_call(ef_r, we_blk, jnp.tile(b1, 8).reshape(1, 128))
    return _sc_edge(a, b, c, edge_index.reshape(2 * _E),
                    jnp.broadcast_to(W2, (_H, _H)), jnp.broadcast_to(b2, (_H,)))


# speedup vs baseline: 6.8846x; 1.0454x over previous
import functools

import jax
import jax.numpy as jnp
from jax import lax
from jax.experimental import pallas as pl
from jax.experimental.pallas import tpu as pltpu
from jax.experimental.pallas import tpu_sc as plsc

_N, _E, _D, _DE, _H = 10000, 320000, 128, 16, 16

_NC, _NS = 2, 16
_NW = _NC * _NS
_CH = 1280
_NCHUNK = _E // _CH
_TMAX = -(-_NCHUNK // _NW)
_SUB = 128
_NSUB = _CH // _SUB
_CROWS = _CH // 8


def _ab_body(x_ref, wa_ref, wb_ref, a_ref, b_ref):
    x = x_ref[...]
    a_ref[...] = jnp.dot(x, wa_ref[...], preferred_element_type=jnp.float32)
    b_ref[...] = jnp.dot(x, wb_ref[...], preferred_element_type=jnp.float32)


_ab_call = pl.pallas_call(
    _ab_body,
    out_shape=[
        jax.ShapeDtypeStruct((_N, _H), jnp.float32),
        jax.ShapeDtypeStruct((_N, _H), jnp.float32),
    ],
)

_BC = 4000


def _c_body(ef_ref, w_ref, b_ref, c_ref):
    c_ref[...] = (
        jnp.dot(ef_ref[...], w_ref[...], preferred_element_type=jnp.float32)
        + b_ref[...]
    )


_c_call = pl.pallas_call(
    _c_body,
    grid=(_E // 8 // _BC,),
    in_specs=[
        pl.BlockSpec((_BC, 128), lambda i: (i, 0)),
        pl.BlockSpec((128, 128), lambda i: (0, 0)),
        pl.BlockSpec((1, 128), lambda i: (0, 0)),
    ],
    out_specs=pl.BlockSpec((_BC, 128), lambda i: (i, 0)),
    out_shape=jax.ShapeDtypeStruct((_E // 8, 128), jnp.float32),
)

_mesh = plsc.VectorSubcoreMesh(
    core_axis_name="core", subcore_axis_name="subcore",
    num_cores=_NC, num_subcores=_NS,
)


_sc_params = pltpu.CompilerParams(
    needs_layout_passes=False, use_tc_tiling_on_sc=False)


@functools.partial(
    pl.kernel,
    out_type=jax.ShapeDtypeStruct((_E, _H), jnp.float32),
    mesh=_mesh,
    compiler_params=_sc_params,
    scratch_types=[
        pltpu.VMEM((_CH,), jnp.int32),
        pltpu.VMEM((_CH,), jnp.int32),
        pltpu.VMEM((_SUB, _H), jnp.float32),
        pltpu.VMEM((_SUB, _H), jnp.float32),
        pltpu.SemaphoreType.DMA,
        pltpu.SemaphoreType.DMA,
        pltpu.SemaphoreType.DMA,
    ],
)
def _sc_gather(a_hbm, b_hbm, ei_hbm, s_hbm, io_v, id_v, g0_v, g1_v,
               sem_a, sem_b, sem_o):
    wid = lax.axis_index("subcore") * _NC + lax.axis_index("core")
    gbufs = [g0_v, g1_v]

    def chunk_body(t, carry):
        ci = jnp.minimum(t * _NW + wid, _NCHUNK - 1)
        cbase = pl.multiple_of(ci * _CH, _CH)
        pltpu.sync_copy(ei_hbm.at[pl.ds(cbase, _CH)], io_v)
        pltpu.sync_copy(ei_hbm.at[pl.ds(_E + cbase, _CH)], id_v)

        def fire_a(s):
            sl = pl.ds(s * _SUB, _SUB)
            return pltpu.async_copy(
                a_hbm.at[io_v.at[sl]], gbufs[s % 2], sem_a)

        desc_a = {0: fire_a(0)}
        desc_o = {}
        for s in range(_NSUB):
            if s + 1 < _NSUB:
                if s >= 1:
                    desc_o.pop(s - 1).wait()
                desc_a[s + 1] = fire_a(s + 1)
            desc_a.pop(s).wait()
            sl = pl.ds(s * _SUB, _SUB)
            pltpu.async_copy(
                b_hbm.at[id_v.at[sl]], gbufs[s % 2], sem_b,
                add=True).wait()
            desc_o[s] = pltpu.async_copy(
                gbufs[s % 2], s_hbm.at[pl.ds(cbase + s * _SUB, _SUB)],
                sem_o)
        for s in sorted(desc_o):
            desc_o.pop(s).wait()

        return carry

    lax.fori_loop(0, _TMAX, chunk_body, 0)


@functools.partial(
    pl.kernel,
    out_type=jax.ShapeDtypeStruct((_E,), jnp.float32),
    mesh=_mesh,
    compiler_params=_sc_params,
    scratch_types=[
        pltpu.VMEM((_CH, _H), jnp.float32),
        pltpu.VMEM((_CH, _H), jnp.float32),
        pltpu.VMEM((_CROWS, 128), jnp.float32),
        pltpu.VMEM((_CROWS, 128), jnp.float32),
        pltpu.VMEM((_CH,), jnp.float32),
        pltpu.VMEM((_H, _H), jnp.float32),
        pltpu.VMEM((_H,), jnp.float32),
        pltpu.SemaphoreType.DMA,
        pltpu.SemaphoreType.DMA,
    ],
)
def _sc_tail(s_hbm, c_hbm, w2_hbm, b2_hbm, out_hbm,
             sv0_v, sv1_v, rc0_v, rc1_v, o_v, w2_v, b2_v, sem_s, sem_c):
    wid = lax.axis_index("subcore") * _NC + lax.axis_index("core")
    pltpu.sync_copy(w2_hbm, w2_v)
    pltpu.sync_copy(b2_hbm, b2_v)
    w2rows = [w2_v[j, :] for j in range(_H)]
    b2vec = b2_v[:]
    lane = lax.iota(jnp.int32, 16)
    rowoff = lane // 8
    ccols = [(lane % 8) * 16 + j for j in range(_H)]
    svbufs = [sv0_v, sv1_v]
    rcbufs = [rc0_v, rc1_v]
    cis = [jnp.minimum(t * _NW + wid, _NCHUNK - 1) for t in range(_TMAX)]

    def fire(t):
        ci = cis[t]
        cbase = pl.multiple_of(ci * _CH, _CH)
        crow = pl.multiple_of(ci * _CROWS, _CROWS)
        return (
            pltpu.async_copy(
                s_hbm.at[pl.ds(cbase, _CH)], svbufs[t % 2], sem_s),
            pltpu.async_copy(
                c_hbm.at[pl.ds(crow, _CROWS)], rcbufs[t % 2], sem_c),
        )

    desc = fire(0)
    for t in range(_TMAX):
        nxt = fire(t + 1) if t + 1 < _TMAX else None
        cs, cc = desc
        cs.wait()
        cc.wait()
        sv_v = svbufs[t % 2]
        rc_v = rcbufs[t % 2]

        def group_body(g, gcarry, sv_v=sv_v, rc_v=rc_v):
            ridx = g * 16 + lane
            crow16 = g * 2 + rowoff
            acc = b2vec
            for j in range(_H):
                cidx = jnp.full((16,), j, jnp.int32)
                hs = plsc.load_gather(sv_v, [ridx, cidx])
                hc = plsc.load_gather(rc_v, [crow16, ccols[j]])
                h = hs + hc
                acc = acc + jnp.maximum(h, h * 0.01) * w2rows[j]
            o_v[pl.ds(g * 16, 16)] = jnp.maximum(acc, acc * 0.01)
            return gcarry

        lax.fori_loop(0, _CH // 16, group_body, 0)
        cbase = pl.multiple_of(cis[t] * _CH, _CH)
        pltpu.sync_copy(o_v, out_hbm.at[pl.ds(cbase, _CH)])
        desc = nxt


def kernel(x, edge_index, edge_features, W1, b1, W2, b2):
    wa = W1[:_D]
    wb = W1[_D:2 * _D]
    we = W1[2 * _D:]
    a, b = _ab_call(x, wa, wb)
    ef_r = edge_features.reshape(_E // 8, 8 * _DE)
    we_blk = jnp.kron(jnp.eye(8, dtype=jnp.float32), we)
    c = _c_call(ef_r, we_blk, jnp.tile(b1, 8).reshape(1, 128))
    s = _sc_gather(a, b, edge_index.reshape(2 * _E))
    return _sc_tail(s, c,
                    jnp.broadcast_to(W2, (_H, _H)), jnp.broadcast_to(b2, (_H,)))


# speedup vs baseline: 9.0259x; 1.3110x over previous
import functools

import jax
import jax.numpy as jnp
from jax import lax
from jax.experimental import pallas as pl
from jax.experimental.pallas import tpu as pltpu
from jax.experimental.pallas import tpu_sc as plsc

_N, _E, _D, _DE, _H = 10000, 320000, 128, 16, 16

_NC, _NS = 2, 16
_NW = _NC * _NS
_CH = 1280
_NCHUNK = _E // _CH
_TMAX = -(-_NCHUNK // _NW)
_SUB = 128
_NSUB = _CH // _SUB
_CROWS = _CH // 8


def _ab_body(x_ref, wa_ref, wb_ref, a_ref, b_ref):
    x = x_ref[...]
    a_ref[...] = jnp.dot(x, wa_ref[...], preferred_element_type=jnp.float32)
    b_ref[...] = jnp.dot(x, wb_ref[...], preferred_element_type=jnp.float32)


_ab_call = pl.pallas_call(
    _ab_body,
    out_shape=[
        jax.ShapeDtypeStruct((_N, _H), jnp.float32),
        jax.ShapeDtypeStruct((_N, _H), jnp.float32),
    ],
)

_BC = 4000


def _tail_body(ef_ref, s_ref, w_ref, b_ref, w2_ref, b2_ref, r_ref):
    z = (
        jnp.dot(ef_ref[...], w_ref[...], preferred_element_type=jnp.float32)
        + b_ref[...]
        + s_ref[...]
    )
    h = jnp.maximum(z, z * 0.01)
    r = (
        jnp.dot(h, w2_ref[...], preferred_element_type=jnp.float32)
        + b2_ref[...]
    )
    r_ref[...] = jnp.maximum(r, r * 0.01)


_tail_call = pl.pallas_call(
    _tail_body,
    grid=(_E // 8 // _BC,),
    in_specs=[
        pl.BlockSpec((_BC, 128), lambda i: (i, 0)),
        pl.BlockSpec((_BC, 128), lambda i: (i, 0)),
        pl.BlockSpec((128, 128), lambda i: (0, 0)),
        pl.BlockSpec((1, 128), lambda i: (0, 0)),
        pl.BlockSpec((128, 8), lambda i: (0, 0)),
        pl.BlockSpec((1, 8), lambda i: (0, 0)),
    ],
    out_specs=pl.BlockSpec((_BC, 8), lambda i: (i, 0)),
    out_shape=jax.ShapeDtypeStruct((_E // 8, 8), jnp.float32),
)

_mesh = plsc.VectorSubcoreMesh(
    core_axis_name="core", subcore_axis_name="subcore",
    num_cores=_NC, num_subcores=_NS,
)


_sc_params = pltpu.CompilerParams(
    needs_layout_passes=False, use_tc_tiling_on_sc=False)


@functools.partial(
    pl.kernel,
    out_type=jax.ShapeDtypeStruct((_E, _H), jnp.float32),
    mesh=_mesh,
    compiler_params=_sc_params,
    scratch_types=[
        pltpu.VMEM((_CH,), jnp.int32),
        pltpu.VMEM((_CH,), jnp.int32),
        pltpu.VMEM((_SUB, _H), jnp.float32),
        pltpu.VMEM((_SUB, _H), jnp.float32),
        pltpu.VMEM((_SUB, _H), jnp.float32),
        pltpu.VMEM((_SUB, _H), jnp.float32),
        pltpu.SemaphoreType.DMA,
        pltpu.SemaphoreType.DMA,
        pltpu.SemaphoreType.DMA,
        pltpu.SemaphoreType.DMA,
        pltpu.SemaphoreType.DMA,
        pltpu.SemaphoreType.DMA,
    ],
)
def _sc_gather(a_hbm, b_hbm, ei_hbm, s_hbm, io_v, id_v, g0_v, g1_v, g2_v,
               g3_v, sem_a0, sem_a1, sem_b0, sem_b1, sem_o0, sem_o1):
    wid = lax.axis_index("subcore") * _NC + lax.axis_index("core")
    gbufs = [g0_v, g1_v, g2_v, g3_v]
    sems_a = [sem_a0, sem_a1]
    sems_b = [sem_b0, sem_b1]
    sems_o = [sem_o0, sem_o1]

    def chunk_body(t, carry):
        ci = jnp.minimum(t * _NW + wid, _NCHUNK - 1)
        cbase = pl.multiple_of(ci * _CH, _CH)
        pltpu.sync_copy(ei_hbm.at[pl.ds(cbase, _CH)], io_v)
        pltpu.sync_copy(ei_hbm.at[pl.ds(_E + cbase, _CH)], id_v)

        desc_a, desc_b, desc_o = {}, {}, {}

        def fire_a(s):
            if 0 <= s < _NSUB:
                if s - 4 >= 0:
                    desc_o.pop(s - 4).wait()
                desc_a[s] = pltpu.async_copy(
                    a_hbm.at[io_v.at[pl.ds(s * _SUB, _SUB)]],
                    gbufs[s % 4], sems_a[s % 2])

        def advance_b(s):
            if 0 <= s < _NSUB:
                desc_a.pop(s).wait()
                desc_b[s] = pltpu.async_copy(
                    b_hbm.at[id_v.at[pl.ds(s * _SUB, _SUB)]],
                    gbufs[s % 4], sems_b[s % 2], add=True)

        def advance_o(s):
            if 0 <= s < _NSUB:
                desc_b.pop(s).wait()
                desc_o[s] = pltpu.async_copy(
                    gbufs[s % 4],
                    s_hbm.at[pl.ds(cbase + s * _SUB, _SUB)], sems_o[s % 2])

        fire_a(0)
        fire_a(1)
        for s in range(_NSUB):
            advance_b(s)
            fire_a(s + 2)
            advance_o(s - 1)
        advance_o(_NSUB - 1)
        for s in sorted(desc_o):
            desc_o.pop(s).wait()

        return carry

    lax.fori_loop(0, _TMAX, chunk_body, 0)


def kernel(x, edge_index, edge_features, W1, b1, W2, b2):
    wa = W1[:_D]
    wb = W1[_D:2 * _D]
    we = W1[2 * _D:]
    a, b = _ab_call(x, wa, wb)
    ef_r = edge_features.reshape(_E // 8, 8 * _DE)
    we_blk = jnp.kron(jnp.eye(8, dtype=jnp.float32), we)
    w2_blk = jnp.kron(jnp.eye(8, dtype=jnp.float32), W2)
    s = _sc_gather(a, b, edge_index.reshape(2 * _E))
    res8 = _tail_call(ef_r, s.reshape(_E // 8, 128), we_blk,
                      jnp.tile(b1, 8).reshape(1, 128), w2_blk,
                      jnp.broadcast_to(b2, (1, 8)))
    return res8.reshape(_E)
